# Initial kernel scaffold; baseline (speedup 1.0000x reference)
import jax
import jax.numpy as jnp
from jax.experimental import pallas as pl


def kernel(feats, x, last_update, edge_index, t, msg, time_w, time_b, Wq, bq, Wk, bk, Wv, bv, We, Wskip, bskip):
    raise NotImplementedError("write your pallas kernel here")


# speedup vs baseline: 4.3054x; 4.3054x over previous
import jax
import jax.numpy as jnp
import numpy as np
from jax.experimental import pallas as pl

N = 10000
E = 160000
HEADS = 8
C = 32
OUT = 256
TIME_DIM = 32
MSG_DIM = 16
NB = 1000
EB = 2000

_S_SUM = jnp.asarray(
    np.repeat(np.eye(HEADS, dtype=np.float32), C, axis=0) / np.sqrt(C))
_S_EXP = jnp.asarray(np.repeat(np.eye(HEADS, dtype=np.float32), C, axis=1))


def _proj_kernel(h_ref, wq_ref, bq_ref, wk_ref, bk_ref, wv_ref, bv_ref,
                 ws_ref, bs_ref, q_ref, k_ref, v_ref, s_ref):
    h = h_ref[...]
    q_ref[...] = jnp.dot(h, wq_ref[...], preferred_element_type=jnp.float32) + bq_ref[...]
    k_ref[...] = jnp.dot(h, wk_ref[...], preferred_element_type=jnp.float32) + bk_ref[...]
    v_ref[...] = jnp.dot(h, wv_ref[...], preferred_element_type=jnp.float32) + bv_ref[...]
    s_ref[...] = jnp.dot(h, ws_ref[...], preferred_element_type=jnp.float32) + bs_ref[...]


def _edge_kernel(qi_ref, ks_ref, vs_ref, lu_ref, t_ref, msg_ref,
                 wt_ref, wm_ref, tw_ref, tb_ref, ssum_ref,
                 alpha_ref, vj_ref):
    rel = lu_ref[...] - t_ref[...]
    enc = jnp.cos(rel * tw_ref[...] + tb_ref[...])
    e = (jnp.dot(enc, wt_ref[...], preferred_element_type=jnp.float32)
         + jnp.dot(msg_ref[...], wm_ref[...], preferred_element_type=jnp.float32))
    kj = ks_ref[...] + e
    vj_ref[...] = vs_ref[...] + e
    alpha_ref[...] = jnp.dot(qi_ref[...] * kj, ssum_ref[...],
                             preferred_element_type=jnp.float32)


def _exp_kernel(a_ref, m_ref, p_ref):
    p_ref[...] = jnp.exp(a_ref[...] - m_ref[...])


def _norm_kernel(p_ref, d_ref, vj_ref, sexp_ref, out_ref):
    w = p_ref[...] / (d_ref[...] + 1e-16)
    out_ref[...] = vj_ref[...] * jnp.dot(w, sexp_ref[...],
                                         preferred_element_type=jnp.float32)


def kernel(feats, x, last_update, edge_index, t, msg, time_w, time_b,
           Wq, bq, Wk, bk, Wv, bv, We, Wskip, bskip):
    src = edge_index[0]
    dst = edge_index[1]
    h = jnp.concatenate([x, feats], axis=1)

    full2 = lambda shape: pl.BlockSpec(shape, lambda i: (0, 0))
    nblk = pl.BlockSpec((NB, OUT), lambda i: (i, 0))
    eblk = pl.BlockSpec((EB, OUT), lambda i: (i, 0))
    ablk = pl.BlockSpec((EB, HEADS), lambda i: (i, 0))

    q, k, v, skip = pl.pallas_call(
        _proj_kernel,
        grid=(N // NB,),
        in_specs=[pl.BlockSpec((NB, 256), lambda i: (i, 0)),
                  full2((256, OUT)), full2((1, OUT)),
                  full2((256, OUT)), full2((1, OUT)),
                  full2((256, OUT)), full2((1, OUT)),
                  full2((256, OUT)), full2((1, OUT))],
        out_specs=[nblk, nblk, nblk, nblk],
        out_shape=[jax.ShapeDtypeStruct((N, OUT), jnp.float32)] * 4,
    )(h, Wq, bq.reshape(1, OUT), Wk, bk.reshape(1, OUT),
      Wv, bv.reshape(1, OUT), Wskip, bskip.reshape(1, OUT))

    q_i = jnp.take(q, dst, axis=0)
    k_s = jnp.take(k, src, axis=0)
    v_s = jnp.take(v, src, axis=0)
    lu_s = jnp.take(last_update, src).reshape(E, 1)

    alpha, v_j = pl.pallas_call(
        _edge_kernel,
        grid=(E // EB,),
        in_specs=[eblk, eblk, eblk,
                  pl.BlockSpec((EB, 1), lambda i: (i, 0)),
                  pl.BlockSpec((EB, 1), lambda i: (i, 0)),
                  pl.BlockSpec((EB, MSG_DIM), lambda i: (i, 0)),
                  full2((TIME_DIM, OUT)), full2((MSG_DIM, OUT)),
                  full2((1, TIME_DIM)), full2((1, TIME_DIM)),
                  full2((OUT, HEADS))],
        out_specs=[ablk, eblk],
        out_shape=[jax.ShapeDtypeStruct((E, HEADS), jnp.float32),
                   jax.ShapeDtypeStruct((E, OUT), jnp.float32)],
    )(q_i, k_s, v_s, lu_s, t.reshape(E, 1), msg,
      We[:TIME_DIM], We[TIME_DIM:], time_w, time_b.reshape(1, TIME_DIM), _S_SUM)

    amax = jax.ops.segment_max(alpha, dst, num_segments=N)
    p = pl.pallas_call(
        _exp_kernel,
        grid=(E // EB,),
        in_specs=[ablk, ablk],
        out_specs=ablk,
        out_shape=jax.ShapeDtypeStruct((E, HEADS), jnp.float32),
    )(alpha, jnp.take(amax, dst, axis=0))

    asum = jax.ops.segment_sum(p, dst, num_segments=N)
    out_e = pl.pallas_call(
        _norm_kernel,
        grid=(E // EB,),
        in_specs=[ablk, ablk, eblk, full2((HEADS, OUT))],
        out_specs=eblk,
        out_shape=jax.ShapeDtypeStruct((E, OUT), jnp.float32),
    )(p, jnp.take(asum, dst, axis=0), v_j, _S_EXP)

    out = jax.ops.segment_sum(out_e, dst, num_segments=N)
    return out + skip


# speedup vs baseline: 5.0033x; 1.1621x over previous
import jax
import jax.numpy as jnp
import numpy as np
from jax.experimental import pallas as pl

N = 10000
E = 160000
HEADS = 8
C = 32
OUT = 256
TIME_DIM = 32
MSG_DIM = 16
NB = 1000
EB = 2000

_S_SUM = jnp.asarray(
    np.repeat(np.eye(HEADS, dtype=np.float32), C, axis=0) / np.sqrt(C))
_S_EXP = jnp.asarray(np.repeat(np.eye(HEADS, dtype=np.float32), C, axis=1))


def _proj_kernel(h_ref, wq_ref, bq_ref, wk_ref, bk_ref, wv_ref, bv_ref,
                 ws_ref, bs_ref, q_ref, k_ref, v_ref, s_ref):
    h = h_ref[...]
    q_ref[...] = jnp.dot(h, wq_ref[...], preferred_element_type=jnp.float32) + bq_ref[...]
    k_ref[...] = jnp.dot(h, wk_ref[...], preferred_element_type=jnp.float32) + bk_ref[...]
    v_ref[...] = jnp.dot(h, wv_ref[...], preferred_element_type=jnp.float32) + bv_ref[...]
    s_ref[...] = jnp.dot(h, ws_ref[...], preferred_element_type=jnp.float32) + bs_ref[...]


def _edge_kernel(qi_ref, ks_ref, vs_ref, lu_ref, t_ref, msg_ref,
                 wt_ref, wm_ref, tw_ref, tb_ref, ssum_ref,
                 alpha_ref, vj_ref):
    rel = lu_ref[...] - t_ref[...]
    enc = jnp.cos(rel * tw_ref[...] + tb_ref[...])
    e = (jnp.dot(enc, wt_ref[...], preferred_element_type=jnp.float32)
         + jnp.dot(msg_ref[...], wm_ref[...], preferred_element_type=jnp.float32))
    kj = ks_ref[...] + e
    vj_ref[...] = vs_ref[...] + e
    alpha_ref[...] = jnp.dot(qi_ref[...] * kj, ssum_ref[...],
                             preferred_element_type=jnp.float32)


def _wv_kernel(a_ref, m_ref, vj_ref, sexp_ref, out_ref):
    p = jnp.exp(a_ref[...] - m_ref[...])
    out_ref[:, :OUT] = vj_ref[...] * jnp.dot(p, sexp_ref[...],
                                             preferred_element_type=jnp.float32)
    out_ref[:, OUT:] = p


def _final_kernel(up_ref, skip_ref, sexp_ref, out_ref):
    d = jnp.dot(up_ref[:, OUT:], sexp_ref[...],
                preferred_element_type=jnp.float32)
    out_ref[...] = up_ref[:, :OUT] / (d + 1e-16) + skip_ref[...]


def kernel(feats, x, last_update, edge_index, t, msg, time_w, time_b,
           Wq, bq, Wk, bk, Wv, bv, We, Wskip, bskip):
    src = edge_index[0]
    dst = edge_index[1]
    h = jnp.concatenate([x, feats], axis=1)

    full2 = lambda shape: pl.BlockSpec(shape, lambda i: (0, 0))
    nblk = pl.BlockSpec((NB, OUT), lambda i: (i, 0))
    eblk = pl.BlockSpec((EB, OUT), lambda i: (i, 0))
    ablk = pl.BlockSpec((EB, HEADS), lambda i: (i, 0))

    q, k, v, skip = pl.pallas_call(
        _proj_kernel,
        grid=(N // NB,),
        in_specs=[pl.BlockSpec((NB, 256), lambda i: (i, 0)),
                  full2((256, OUT)), full2((1, OUT)),
                  full2((256, OUT)), full2((1, OUT)),
                  full2((256, OUT)), full2((1, OUT)),
                  full2((256, OUT)), full2((1, OUT))],
        out_specs=[nblk, nblk, nblk, nblk],
        out_shape=[jax.ShapeDtypeStruct((N, OUT), jnp.float32)] * 4,
    )(h, Wq, bq.reshape(1, OUT), Wk, bk.reshape(1, OUT),
      Wv, bv.reshape(1, OUT), Wskip, bskip.reshape(1, OUT))

    q_i = jnp.take(q, dst, axis=0)
    k_s = jnp.take(k, src, axis=0)
    v_s = jnp.take(v, src, axis=0)
    lu_s = jnp.take(last_update, src).reshape(E, 1)

    alpha, v_j = pl.pallas_call(
        _edge_kernel,
        grid=(E // EB,),
        in_specs=[eblk, eblk, eblk,
                  pl.BlockSpec((EB, 1), lambda i: (i, 0)),
                  pl.BlockSpec((EB, 1), lambda i: (i, 0)),
                  pl.BlockSpec((EB, MSG_DIM), lambda i: (i, 0)),
                  full2((TIME_DIM, OUT)), full2((MSG_DIM, OUT)),
                  full2((1, TIME_DIM)), full2((1, TIME_DIM)),
                  full2((OUT, HEADS))],
        out_specs=[ablk, eblk],
        out_shape=[jax.ShapeDtypeStruct((E, HEADS), jnp.float32),
                   jax.ShapeDtypeStruct((E, OUT), jnp.float32)],
    )(q_i, k_s, v_s, lu_s, t.reshape(E, 1), msg,
      We[:TIME_DIM], We[TIME_DIM:], time_w, time_b.reshape(1, TIME_DIM), _S_SUM)

    amax = jax.ops.segment_max(alpha, dst, num_segments=N)
    wvp = pl.pallas_call(
        _wv_kernel,
        grid=(E // EB,),
        in_specs=[ablk, ablk, eblk, full2((HEADS, OUT))],
        out_specs=pl.BlockSpec((EB, OUT + HEADS), lambda i: (i, 0)),
        out_shape=jax.ShapeDtypeStruct((E, OUT + HEADS), jnp.float32),
    )(alpha, jnp.take(amax, dst, axis=0), v_j, _S_EXP)

    up = jax.ops.segment_sum(wvp, dst, num_segments=N)
    return pl.pallas_call(
        _final_kernel,
        grid=(N // NB,),
        in_specs=[pl.BlockSpec((NB, OUT + HEADS), lambda i: (i, 0)),
                  nblk, full2((HEADS, OUT))],
        out_specs=nblk,
        out_shape=jax.ShapeDtypeStruct((N, OUT), jnp.float32),
    )(up, skip, _S_EXP)
